# baseline (speedup 1.0000x reference)
import jax
import jax.numpy as jnp
from jax.experimental import pallas as pl

_B, _S, _D = 16, 1024, 256
_ROWS = _B * _S
_BLK = 2048


def _copy_kernel(x_ref, o_ref):
    o_ref[...] = x_ref[...]


def kernel(x, W):
    del W
    flat = x.reshape(_ROWS, _D)
    out = pl.pallas_call(
        _copy_kernel,
        grid=(_ROWS // _BLK,),
        in_specs=[pl.BlockSpec((_BLK, _D), lambda i: (i, 0))],
        out_specs=pl.BlockSpec((_BLK, _D), lambda i: (i, 0)),
        out_shape=jax.ShapeDtypeStruct((_ROWS, _D), x.dtype),
    )(flat)
    return out.reshape(x.shape)


# speedup vs baseline: 1.0074x; 1.0074x over previous
import jax
import jax.numpy as jnp
from jax.experimental import pallas as pl
from jax.experimental.pallas import tpu as pltpu

_B, _S, _D = 16, 1024, 256
_ROWS = _B * _S
_BLK = 2048


def _copy_kernel(x_ref, o_ref):
    o_ref[...] = x_ref[...]


def kernel(x, W):
    del W
    flat = x.reshape(_ROWS, _D)
    out = pl.pallas_call(
        _copy_kernel,
        grid=(_ROWS // _BLK,),
        in_specs=[pl.BlockSpec((_BLK, _D), lambda i: (i, 0))],
        out_specs=pl.BlockSpec((_BLK, _D), lambda i: (i, 0)),
        out_shape=jax.ShapeDtypeStruct((_ROWS, _D), x.dtype),
        compiler_params=pltpu.CompilerParams(
            dimension_semantics=("parallel",),
        ),
    )(flat)
    return out.reshape(x.shape)
